# baseline (speedup 1.0000x reference)
import functools

import jax
import jax.numpy as jnp
from jax.experimental import pallas as pl
from jax.experimental.pallas import tpu as pltpu

_EPS = 1e-12
_BB = 8


def _address(cos, beta, gate, shift, gamma, prev_w):
    b = jax.nn.softplus(beta)
    a = b * cos
    a = a - jnp.max(a, axis=-1, keepdims=True)
    e = jnp.exp(a)
    wt = e / jnp.sum(e, axis=-1, keepdims=True)
    g = jax.nn.sigmoid(gate)
    wt = g * wt + (1.0 - g) * prev_w
    s = shift - jnp.max(shift, axis=-1, keepdims=True)
    s = jnp.exp(s)
    s = s / jnp.sum(s, axis=-1, keepdims=True)
    wm1 = jnp.concatenate([wt[:, 1:], wt[:, :1]], axis=1)
    wp1 = jnp.concatenate([wt[:, -1:], wt[:, :-1]], axis=1)
    wt = s[:, 0:1] * wm1 + s[:, 1:2] * wt + s[:, 2:3] * wp1
    gam = 1.0 + jax.nn.softplus(gamma)
    wt = jnp.exp(gam * jnp.log(wt + _EPS))
    return wt / (jnp.sum(wt, axis=-1, keepdims=True) + _EPS)


def _ntm_body(x_ref, mem_ref, rw_ref, ww_ref, h_ref, wih_ref, whh_ref,
              bih_ref, bhh_ref, pw_ref, pb_ref,
              y_ref, nm_ref, nrw_ref, wtw_ref, *, H):
    M = x_ref.shape[1]
    Bb = x_ref.shape[0]
    mem = mem_ref[...]
    rw = rw_ref[...]
    ww = ww_ref[...]

    read_vec = jnp.concatenate(
        [jnp.dot(rw[b:b + 1, :], mem[b]) for b in range(Bb)], axis=0)

    hT = jnp.transpose(h_ref[...])
    wih = wih_ref[...]
    whh = whh_ref[...]
    bih = bih_ref[...]
    bhh = bhh_ref[...]

    def gru_t(xt, hT):
        giT = jax.lax.dot_general(wih, xt,
                                  (((1,), (1,)), ((), ()))) + bih
        ghT = jax.lax.dot_general(whh, hT,
                                  (((1,), (0,)), ((), ()))) + bhh
        r = jax.nn.sigmoid(giT[:H] + ghT[:H])
        z = jax.nn.sigmoid(giT[H:2 * H] + ghT[H:2 * H])
        n = jnp.tanh(giT[2 * H:] + r * ghT[2 * H:])
        return (1.0 - z) * n + z * hT

    hT = gru_t(x_ref[...], hT)
    hT = gru_t(read_vec, hT)
    coT = jax.lax.dot_general(pw_ref[...], hT,
                              (((1,), (0,)), ((), ()))) + pb_ref[...]

    rl = M + 6
    key_rT = coT[:M]
    key_wT = coT[rl:rl + M]
    eraseT = jax.nn.sigmoid(coT[rl + M + 6:rl + 2 * M + 6])
    addvT = coT[rl + 2 * M + 6:rl + 3 * M + 6]
    y_ref[...] = jnp.transpose(coT[rl + 3 * M + 6:rl + 3 * M + 6 + M])
    sc = jnp.transpose(jnp.concatenate(
        [coT[M:rl], coT[rl + M:rl + M + 6]], axis=0))

    krT = key_rT / (jnp.sqrt(jnp.sum(key_rT * key_rT, axis=0,
                                     keepdims=True)) + _EPS)
    kwT = key_wT / (jnp.sqrt(jnp.sum(key_wT * key_wT, axis=0,
                                     keepdims=True)) + _EPS)

    inv_norm = 1.0 / (jnp.sqrt(jnp.sum(mem * mem, axis=2)) + _EPS)

    dcols = []
    for b in range(Bb):
        keys2 = jnp.concatenate([krT[:, b:b + 1], kwT[:, b:b + 1]], axis=1)
        dcols.append(jnp.dot(mem[b], keys2))
    d3 = jnp.swapaxes(jnp.stack(dcols, axis=0), 1, 2)
    cos_r = d3[:, 0, :] * inv_norm
    cos_w = d3[:, 1, :] * inv_norm

    new_rw = _address(cos_r, sc[:, 0:1], sc[:, 1:2], sc[:, 2:5], sc[:, 5:6], rw)
    wt_w = _address(cos_w, sc[:, 6:7], sc[:, 7:8], sc[:, 8:11], sc[:, 11:12], ww)
    nrw_ref[...] = new_rw
    wtw_ref[...] = wt_w

    erase = jnp.transpose(eraseT)
    addv = jnp.transpose(addvT)
    nm_ref[...] = mem + wt_w[:, :, None] * (addv[:, None, :]
                                            - erase[:, None, :] * mem)


def kernel(x, memory, read_w, write_w, hidden, w_ih, w_hh, b_ih, b_hh,
           proj_w, proj_b):
    B, N, M = memory.shape
    H = hidden.shape[-1]
    P = 5 * M + 12
    Pp = ((P + 7) // 8) * 8
    pw = jnp.pad(proj_w, ((0, Pp - P), (0, 0)))
    pb = jnp.pad(proj_b, (0, Pp - P))[:, None]

    grid = (B // _BB,)
    body = functools.partial(_ntm_body, H=H)
    y, new_mem, nrw, wtw = pl.pallas_call(
        body,
        grid=grid,
        in_specs=[
            pl.BlockSpec((_BB, M), lambda i: (i, 0)),
            pl.BlockSpec((_BB, N, M), lambda i: (i, 0, 0)),
            pl.BlockSpec((_BB, N), lambda i: (i, 0)),
            pl.BlockSpec((_BB, N), lambda i: (i, 0)),
            pl.BlockSpec((_BB, H), lambda i: (i, 0)),
            pl.BlockSpec((3 * H, M), lambda i: (0, 0)),
            pl.BlockSpec((3 * H, H), lambda i: (0, 0)),
            pl.BlockSpec((3 * H, 1), lambda i: (0, 0)),
            pl.BlockSpec((3 * H, 1), lambda i: (0, 0)),
            pl.BlockSpec((Pp, H), lambda i: (0, 0)),
            pl.BlockSpec((Pp, 1), lambda i: (0, 0)),
        ],
        out_specs=[
            pl.BlockSpec((_BB, M), lambda i: (i, 0)),
            pl.BlockSpec((_BB, N, M), lambda i: (i, 0, 0)),
            pl.BlockSpec((_BB, N), lambda i: (i, 0)),
            pl.BlockSpec((_BB, N), lambda i: (i, 0)),
        ],
        out_shape=[
            jax.ShapeDtypeStruct((B, M), jnp.float32),
            jax.ShapeDtypeStruct((B, N, M), jnp.float32),
            jax.ShapeDtypeStruct((B, N), jnp.float32),
            jax.ShapeDtypeStruct((B, N), jnp.float32),
        ],
        compiler_params=pltpu.CompilerParams(
            dimension_semantics=("parallel",),
            vmem_limit_bytes=56 * 1024 * 1024,
        ),
    )(x, memory, read_w[:, 0, :], write_w[:, 0, :], hidden[0],
      w_ih, w_hh, b_ih[:, None], b_hh[:, None], pw, pb)

    return y, new_mem, nrw[:, None, :], wtw[:, None, :]


# speedup vs baseline: 1.1189x; 1.1189x over previous
import functools

import jax
import jax.numpy as jnp
from jax.experimental import pallas as pl
from jax.experimental.pallas import tpu as pltpu

_EPS = 1e-12
_BB = 8


def _address(cos, beta, gate, shift, gamma, prev_w):
    b = jax.nn.softplus(beta)
    a = b * cos
    a = a - jnp.max(a, axis=-1, keepdims=True)
    e = jnp.exp(a)
    wt = e / jnp.sum(e, axis=-1, keepdims=True)
    g = jax.nn.sigmoid(gate)
    wt = g * wt + (1.0 - g) * prev_w
    s = shift - jnp.max(shift, axis=-1, keepdims=True)
    s = jnp.exp(s)
    s = s / jnp.sum(s, axis=-1, keepdims=True)
    wm1 = jnp.concatenate([wt[:, 1:], wt[:, :1]], axis=1)
    wp1 = jnp.concatenate([wt[:, -1:], wt[:, :-1]], axis=1)
    wt = s[:, 0:1] * wm1 + s[:, 1:2] * wt + s[:, 2:3] * wp1
    gam = 1.0 + jax.nn.softplus(gamma)
    wt = jnp.exp(gam * jnp.log(wt + _EPS))
    return wt / (jnp.sum(wt, axis=-1, keepdims=True) + _EPS)


def _ntm_body(x_ref, mem_ref, rw_ref, ww_ref, h_ref, wih_ref, whh_ref,
              bih_ref, bhh_ref, pwin_ref, pbin_ref,
              y_ref, nm_ref, nrw_ref, wtw_ref,
              wiht_s, whht_s, pw_s, pb_s, *, H, PP):
    M = x_ref.shape[1]
    Bb = x_ref.shape[0]
    rl = M + 6
    P = 5 * M + 12

    @pl.when(pl.program_id(0) == 0)
    def _prep():
        wiht_s[...] = jnp.transpose(wih_ref[...])
        whht_s[...] = jnp.transpose(whh_ref[...])
        pwin = pwin_ref[...]
        t = jnp.concatenate([
            pwin[0:M], pwin[rl:rl + M],
            pwin[rl + M + 6:rl + 2 * M + 6],
            pwin[rl + 2 * M + 6:rl + 3 * M + 6],
            pwin[rl + 3 * M + 6:rl + 4 * M + 6],
            pwin[M:rl], pwin[rl + M:rl + M + 6],
            jnp.zeros((PP - P, H), jnp.float32)], axis=0)
        pw_s[...] = jnp.transpose(t)
        pbin = pbin_ref[...]
        tb = jnp.concatenate([
            pbin[0:M], pbin[rl:rl + M],
            pbin[rl + M + 6:rl + 2 * M + 6],
            pbin[rl + 2 * M + 6:rl + 3 * M + 6],
            pbin[rl + 3 * M + 6:rl + 4 * M + 6],
            pbin[M:rl], pbin[rl + M:rl + M + 6],
            jnp.zeros((PP - P, 1), jnp.float32)], axis=0)
        pb_s[...] = jnp.transpose(tb)

    mem = mem_ref[...]
    rw = rw_ref[...]
    ww = ww_ref[...]

    read_vec = jnp.concatenate(
        [jnp.dot(rw[b:b + 1, :], mem[b]) for b in range(Bb)], axis=0)

    h = h_ref[...]
    wih = wiht_s[...]
    whh = whht_s[...]
    bih = bih_ref[...]
    bhh = bhh_ref[...]

    def gru(xt, h):
        gi = jnp.dot(xt, wih) + bih
        gh = jnp.dot(h, whh) + bhh
        r = jax.nn.sigmoid(gi[:, :H] + gh[:, :H])
        z = jax.nn.sigmoid(gi[:, H:2 * H] + gh[:, H:2 * H])
        n = jnp.tanh(gi[:, 2 * H:] + r * gh[:, 2 * H:])
        return (1.0 - z) * n + z * h

    h = gru(x_ref[...], h)
    h = gru(read_vec, h)
    co = jnp.dot(h, pw_s[...]) + pb_s[...]

    key_r = co[:, :M]
    key_w = co[:, M:2 * M]
    erase = jax.nn.sigmoid(co[:, 2 * M:3 * M])
    addv = co[:, 3 * M:4 * M]
    y_ref[...] = co[:, 4 * M:5 * M]
    sc = co[:, 5 * M:5 * M + 12]

    inv_norm = 1.0 / (jnp.sqrt(jnp.sum(mem * mem, axis=2)) + _EPS)
    kr = key_r / (jnp.sqrt(jnp.sum(key_r * key_r, axis=1, keepdims=True)) + _EPS)
    kw = key_w / (jnp.sqrt(jnp.sum(key_w * key_w, axis=1, keepdims=True)) + _EPS)

    dcols = []
    for b in range(Bb):
        keys2 = jnp.concatenate([kr[b:b + 1, :], kw[b:b + 1, :]], axis=0)
        dcols.append(jax.lax.dot_general(
            mem[b], keys2, (((1,), (1,)), ((), ()))))
    d3 = jnp.swapaxes(jnp.stack(dcols, axis=0), 1, 2)
    cos_r = d3[:, 0, :] * inv_norm
    cos_w = d3[:, 1, :] * inv_norm

    new_rw = _address(cos_r, sc[:, 0:1], sc[:, 1:2], sc[:, 2:5], sc[:, 5:6], rw)
    wt_w = _address(cos_w, sc[:, 6:7], sc[:, 7:8], sc[:, 8:11], sc[:, 11:12], ww)
    nrw_ref[...] = new_rw
    wtw_ref[...] = wt_w

    nm_ref[...] = mem + wt_w[:, :, None] * (addv[:, None, :]
                                            - erase[:, None, :] * mem)


def kernel(x, memory, read_w, write_w, hidden, w_ih, w_hh, b_ih, b_hh,
           proj_w, proj_b):
    B, N, M = memory.shape
    H = hidden.shape[-1]
    P = 5 * M + 12
    PP = ((P + 127) // 128) * 128
    Pp = ((P + 7) // 8) * 8
    pwin = jnp.pad(proj_w, ((0, Pp - P), (0, 0)))
    pbin = jnp.pad(proj_b, (0, Pp - P))[:, None]

    grid = (B // _BB,)
    body = functools.partial(_ntm_body, H=H, PP=PP)
    y, new_mem, nrw, wtw = pl.pallas_call(
        body,
        grid=grid,
        in_specs=[
            pl.BlockSpec((_BB, M), lambda i: (i, 0)),
            pl.BlockSpec((_BB, N, M), lambda i: (i, 0, 0)),
            pl.BlockSpec((_BB, N), lambda i: (i, 0)),
            pl.BlockSpec((_BB, N), lambda i: (i, 0)),
            pl.BlockSpec((_BB, H), lambda i: (i, 0)),
            pl.BlockSpec((3 * H, M), lambda i: (0, 0)),
            pl.BlockSpec((3 * H, H), lambda i: (0, 0)),
            pl.BlockSpec((1, 3 * H), lambda i: (0, 0)),
            pl.BlockSpec((1, 3 * H), lambda i: (0, 0)),
            pl.BlockSpec((Pp, H), lambda i: (0, 0)),
            pl.BlockSpec((Pp, 1), lambda i: (0, 0)),
        ],
        out_specs=[
            pl.BlockSpec((_BB, M), lambda i: (i, 0)),
            pl.BlockSpec((_BB, N, M), lambda i: (i, 0, 0)),
            pl.BlockSpec((_BB, N), lambda i: (i, 0)),
            pl.BlockSpec((_BB, N), lambda i: (i, 0)),
        ],
        out_shape=[
            jax.ShapeDtypeStruct((B, M), jnp.float32),
            jax.ShapeDtypeStruct((B, N, M), jnp.float32),
            jax.ShapeDtypeStruct((B, N), jnp.float32),
            jax.ShapeDtypeStruct((B, N), jnp.float32),
        ],
        scratch_shapes=[
            pltpu.VMEM((M, 3 * H), jnp.float32),
            pltpu.VMEM((H, 3 * H), jnp.float32),
            pltpu.VMEM((H, PP), jnp.float32),
            pltpu.VMEM((1, PP), jnp.float32),
        ],
        compiler_params=pltpu.CompilerParams(
            dimension_semantics=("arbitrary",),
            vmem_limit_bytes=56 * 1024 * 1024,
        ),
    )(x, memory, read_w[:, 0, :], write_w[:, 0, :], hidden[0],
      w_ih, w_hh, b_ih[None], b_hh[None], pwin, pbin)

    return y, new_mem, nrw[:, None, :], wtw[:, None, :]


# speedup vs baseline: 1.1503x; 1.0281x over previous
import functools

import jax
import jax.numpy as jnp
from jax.experimental import pallas as pl
from jax.experimental.pallas import tpu as pltpu

_EPS = 1e-12
_BB = 8


def _address(cos, beta, gate, shift, gamma, prev_w):
    b = jax.nn.softplus(beta)
    a = b * cos
    a = a - jnp.max(a, axis=-1, keepdims=True)
    e = jnp.exp(a)
    wt = e / jnp.sum(e, axis=-1, keepdims=True)
    g = jax.nn.sigmoid(gate)
    wt = g * wt + (1.0 - g) * prev_w
    s = shift - jnp.max(shift, axis=-1, keepdims=True)
    s = jnp.exp(s)
    s = s / jnp.sum(s, axis=-1, keepdims=True)
    wm1 = jnp.concatenate([wt[:, 1:], wt[:, :1]], axis=1)
    wp1 = jnp.concatenate([wt[:, -1:], wt[:, :-1]], axis=1)
    wt = s[:, 0:1] * wm1 + s[:, 1:2] * wt + s[:, 2:3] * wp1
    gam = 1.0 + jax.nn.softplus(gamma)
    wt = jnp.exp(gam * jnp.log(wt + _EPS))
    return wt / (jnp.sum(wt, axis=-1, keepdims=True) + _EPS)


def _ntm_body(x_ref, mem_ref, rw_ref, ww_ref, h_ref, wih_ref, whh_ref,
              bih_ref, bhh_ref, pwin_ref, pbin_ref,
              y_ref, nm_ref, nrw_ref, wtw_ref,
              wiht_s, whht_s, pw_s, pb_s, *, H, PP):
    M = x_ref.shape[1]
    Bb = x_ref.shape[0]
    rl = M + 6
    P = 5 * M + 12

    @pl.when(pl.program_id(0) == 0)
    def _prep():
        wiht_s[...] = jnp.transpose(wih_ref[...])
        whht_s[...] = jnp.transpose(whh_ref[...])
        pwin = pwin_ref[...]
        t = jnp.concatenate([
            pwin[0:M], pwin[rl:rl + M],
            pwin[rl + M + 6:rl + 2 * M + 6],
            pwin[rl + 2 * M + 6:rl + 3 * M + 6],
            pwin[rl + 3 * M + 6:rl + 4 * M + 6],
            pwin[M:rl], pwin[rl + M:rl + M + 6],
            jnp.zeros((PP - P, H), jnp.float32)], axis=0)
        pw_s[...] = jnp.transpose(t)
        pbrow = pbin_ref[...]
        pb_s[...] = jnp.concatenate([
            pbrow[:, 0:M], pbrow[:, rl:rl + M],
            pbrow[:, rl + M + 6:rl + 2 * M + 6],
            pbrow[:, rl + 2 * M + 6:rl + 3 * M + 6],
            pbrow[:, rl + 3 * M + 6:rl + 4 * M + 6],
            pbrow[:, M:rl], pbrow[:, rl + M:rl + M + 6],
            jnp.zeros((1, PP - P), jnp.float32)], axis=1)

    mem = mem_ref[...]
    rw = rw_ref[...]
    ww = ww_ref[...]

    read_vec = jnp.concatenate(
        [jnp.dot(rw[b:b + 1, :], mem[b]) for b in range(Bb)], axis=0)

    h = h_ref[...]
    wih = wiht_s[...]
    whh = whht_s[...]
    bih = bih_ref[...]
    bhh = bhh_ref[...]

    def gru(xt, h):
        gi = jnp.dot(xt, wih) + bih
        gh = jnp.dot(h, whh) + bhh
        r = jax.nn.sigmoid(gi[:, :H] + gh[:, :H])
        z = jax.nn.sigmoid(gi[:, H:2 * H] + gh[:, H:2 * H])
        n = jnp.tanh(gi[:, 2 * H:] + r * gh[:, 2 * H:])
        return (1.0 - z) * n + z * h

    h = gru(x_ref[...], h)
    h = gru(read_vec, h)
    co = jnp.dot(h, pw_s[...]) + pb_s[...]

    key_r = co[:, :M]
    key_w = co[:, M:2 * M]
    erase = jax.nn.sigmoid(co[:, 2 * M:3 * M])
    addv = co[:, 3 * M:4 * M]
    y_ref[...] = co[:, 4 * M:5 * M]
    sc = co[:, 5 * M:5 * M + 12]

    inv_norm = 1.0 / (jnp.sqrt(jnp.sum(mem * mem, axis=2)) + _EPS)
    kr = key_r / (jnp.sqrt(jnp.sum(key_r * key_r, axis=1, keepdims=True)) + _EPS)
    kw = key_w / (jnp.sqrt(jnp.sum(key_w * key_w, axis=1, keepdims=True)) + _EPS)

    dcols = []
    for b in range(Bb):
        keys2 = jnp.concatenate([kr[b:b + 1, :], kw[b:b + 1, :]], axis=0)
        dcols.append(jax.lax.dot_general(
            mem[b], keys2, (((1,), (1,)), ((), ()))))
    d3 = jnp.swapaxes(jnp.stack(dcols, axis=0), 1, 2)
    cos_r = d3[:, 0, :] * inv_norm
    cos_w = d3[:, 1, :] * inv_norm

    new_rw = _address(cos_r, sc[:, 0:1], sc[:, 1:2], sc[:, 2:5], sc[:, 5:6], rw)
    wt_w = _address(cos_w, sc[:, 6:7], sc[:, 7:8], sc[:, 8:11], sc[:, 11:12], ww)
    nrw_ref[...] = new_rw
    wtw_ref[...] = wt_w

    nm_ref[...] = mem + wt_w[:, :, None] * (addv[:, None, :]
                                            - erase[:, None, :] * mem)


def kernel(x, memory, read_w, write_w, hidden, w_ih, w_hh, b_ih, b_hh,
           proj_w, proj_b):
    B, N, M = memory.shape
    H = hidden.shape[-1]
    P = 5 * M + 12
    PP = ((P + 127) // 128) * 128
    Pp = ((P + 7) // 8) * 8
    pwin = jnp.pad(proj_w, ((0, Pp - P), (0, 0)))
    pbin = jnp.reshape(proj_b, (1, P))

    grid = (B // _BB,)
    body = functools.partial(_ntm_body, H=H, PP=PP)
    y, new_mem, nrw, wtw = pl.pallas_call(
        body,
        grid=grid,
        in_specs=[
            pl.BlockSpec((_BB, M), lambda i: (i, 0)),
            pl.BlockSpec((_BB, N, M), lambda i: (i, 0, 0)),
            pl.BlockSpec((_BB, N), lambda i: (i, 0)),
            pl.BlockSpec((_BB, N), lambda i: (i, 0)),
            pl.BlockSpec((_BB, H), lambda i: (i, 0)),
            pl.BlockSpec((3 * H, M), lambda i: (0, 0)),
            pl.BlockSpec((3 * H, H), lambda i: (0, 0)),
            pl.BlockSpec((1, 3 * H), lambda i: (0, 0)),
            pl.BlockSpec((1, 3 * H), lambda i: (0, 0)),
            pl.BlockSpec((Pp, H), lambda i: (0, 0)),
            pl.BlockSpec((1, P), lambda i: (0, 0)),
        ],
        out_specs=[
            pl.BlockSpec((_BB, M), lambda i: (i, 0)),
            pl.BlockSpec((_BB, N, M), lambda i: (i, 0, 0)),
            pl.BlockSpec((_BB, N), lambda i: (i, 0)),
            pl.BlockSpec((_BB, N), lambda i: (i, 0)),
        ],
        out_shape=[
            jax.ShapeDtypeStruct((B, M), jnp.float32),
            jax.ShapeDtypeStruct((B, N, M), jnp.float32),
            jax.ShapeDtypeStruct((B, N), jnp.float32),
            jax.ShapeDtypeStruct((B, N), jnp.float32),
        ],
        scratch_shapes=[
            pltpu.VMEM((M, 3 * H), jnp.float32),
            pltpu.VMEM((H, 3 * H), jnp.float32),
            pltpu.VMEM((H, PP), jnp.float32),
            pltpu.VMEM((1, PP), jnp.float32),
        ],
        compiler_params=pltpu.CompilerParams(
            dimension_semantics=("arbitrary",),
            vmem_limit_bytes=56 * 1024 * 1024,
        ),
    )(x, memory, jnp.reshape(read_w, (B, N)), jnp.reshape(write_w, (B, N)),
      jnp.reshape(hidden, (B, H)), w_ih, w_hh,
      jnp.reshape(b_ih, (1, 3 * H)), jnp.reshape(b_hh, (1, 3 * H)),
      pwin, pbin)

    return y, new_mem, nrw[:, None, :], wtw[:, None, :]


# speedup vs baseline: 1.1543x; 1.0034x over previous
import functools

import jax
import jax.numpy as jnp
from jax.experimental import pallas as pl
from jax.experimental.pallas import tpu as pltpu

_EPS = 1e-12
_BB = 8


def _address(cos, beta, gate, shift, gamma, prev_w):
    b = jax.nn.softplus(beta)
    a = b * cos
    a = a - jnp.max(a, axis=-1, keepdims=True)
    e = jnp.exp(a)
    wt = e / jnp.sum(e, axis=-1, keepdims=True)
    g = jax.nn.sigmoid(gate)
    wt = g * wt + (1.0 - g) * prev_w
    s = shift - jnp.max(shift, axis=-1, keepdims=True)
    s = jnp.exp(s)
    s = s / jnp.sum(s, axis=-1, keepdims=True)
    wm1 = jnp.concatenate([wt[:, 1:], wt[:, :1]], axis=1)
    wp1 = jnp.concatenate([wt[:, -1:], wt[:, :-1]], axis=1)
    wt = s[:, 0:1] * wm1 + s[:, 1:2] * wt + s[:, 2:3] * wp1
    gam = 1.0 + jax.nn.softplus(gamma)
    wt = jnp.exp(gam * jnp.log(wt + _EPS))
    return wt / (jnp.sum(wt, axis=-1, keepdims=True) + _EPS)


def _ntm_body(x_ref, mem_ref, rw_ref, ww_ref, h_ref, wih_ref, whh_ref,
              bih_ref, bhh_ref, pwin_ref, pbin_ref,
              y_ref, nm_ref, nrw_ref, wtw_ref,
              wiht_s, whht_s, pw_s, pb_s, *, H, PP):
    M = x_ref.shape[1]
    Bb = x_ref.shape[0]
    rl = M + 6
    P = 5 * M + 12

    @pl.when(pl.program_id(0) == 0)
    def _prep():
        wiht_s[...] = jnp.transpose(wih_ref[...])
        whht_s[...] = jnp.transpose(whh_ref[...])
        pwin = pwin_ref[...]
        t = jnp.concatenate([
            pwin[0:M], pwin[rl:rl + M],
            pwin[rl + M + 6:rl + 2 * M + 6],
            pwin[rl + 2 * M + 6:rl + 3 * M + 6],
            pwin[rl + 3 * M + 6:rl + 4 * M + 6],
            pwin[M:rl], pwin[rl + M:rl + M + 6],
            jnp.zeros((PP - P, H), jnp.float32)], axis=0)
        pw_s[...] = jnp.transpose(t)
        pbrow = pbin_ref[...]
        pb_s[...] = jnp.concatenate([
            pbrow[:, 0:M], pbrow[:, rl:rl + M],
            pbrow[:, rl + M + 6:rl + 2 * M + 6],
            pbrow[:, rl + 2 * M + 6:rl + 3 * M + 6],
            pbrow[:, rl + 3 * M + 6:rl + 4 * M + 6],
            pbrow[:, M:rl], pbrow[:, rl + M:rl + M + 6],
            jnp.zeros((1, PP - P), jnp.float32)], axis=1)

    mem = mem_ref[...]
    rw = rw_ref[:, 0, :]
    ww = ww_ref[:, 0, :]

    read_vec = jnp.concatenate(
        [jnp.dot(rw[b:b + 1, :], mem[b]) for b in range(Bb)], axis=0)

    h = h_ref[0]
    wih = wiht_s[...]
    whh = whht_s[...]
    bih = bih_ref[...]
    bhh = bhh_ref[...]

    def gru(xt, h):
        gi = jnp.dot(xt, wih) + bih
        gh = jnp.dot(h, whh) + bhh
        r = jax.nn.sigmoid(gi[:, :H] + gh[:, :H])
        z = jax.nn.sigmoid(gi[:, H:2 * H] + gh[:, H:2 * H])
        n = jnp.tanh(gi[:, 2 * H:] + r * gh[:, 2 * H:])
        return (1.0 - z) * n + z * h

    h = gru(x_ref[...], h)
    h = gru(read_vec, h)
    co = jnp.dot(h, pw_s[...]) + pb_s[...]

    key_r = co[:, :M]
    key_w = co[:, M:2 * M]
    erase = jax.nn.sigmoid(co[:, 2 * M:3 * M])
    addv = co[:, 3 * M:4 * M]
    y_ref[...] = co[:, 4 * M:5 * M]
    sc = co[:, 5 * M:5 * M + 12]

    inv_norm = 1.0 / (jnp.sqrt(jnp.sum(mem * mem, axis=2)) + _EPS)
    kr = key_r / (jnp.sqrt(jnp.sum(key_r * key_r, axis=1, keepdims=True)) + _EPS)
    kw = key_w / (jnp.sqrt(jnp.sum(key_w * key_w, axis=1, keepdims=True)) + _EPS)

    dcols = []
    for b in range(Bb):
        keys2 = jnp.concatenate([kr[b:b + 1, :], kw[b:b + 1, :]], axis=0)
        dcols.append(jax.lax.dot_general(
            mem[b], keys2, (((1,), (1,)), ((), ()))))
    d3 = jnp.swapaxes(jnp.stack(dcols, axis=0), 1, 2)
    cos_r = d3[:, 0, :] * inv_norm
    cos_w = d3[:, 1, :] * inv_norm

    new_rw = _address(cos_r, sc[:, 0:1], sc[:, 1:2], sc[:, 2:5], sc[:, 5:6], rw)
    wt_w = _address(cos_w, sc[:, 6:7], sc[:, 7:8], sc[:, 8:11], sc[:, 11:12], ww)
    nrw_ref[...] = new_rw[:, None, :]
    wtw_ref[...] = wt_w[:, None, :]

    nm_ref[...] = mem + wt_w[:, :, None] * (addv[:, None, :]
                                            - erase[:, None, :] * mem)


def kernel(x, memory, read_w, write_w, hidden, w_ih, w_hh, b_ih, b_hh,
           proj_w, proj_b):
    B, N, M = memory.shape
    H = hidden.shape[-1]
    P = 5 * M + 12
    PP = ((P + 127) // 128) * 128
    Pp = ((P + 7) // 8) * 8
    pwin = jnp.pad(proj_w, ((0, Pp - P), (0, 0)))
    pbin = jnp.reshape(proj_b, (1, P))

    grid = (B // _BB,)
    body = functools.partial(_ntm_body, H=H, PP=PP)
    y, new_mem, nrw, wtw = pl.pallas_call(
        body,
        grid=grid,
        in_specs=[
            pl.BlockSpec((_BB, M), lambda i: (i, 0)),
            pl.BlockSpec((_BB, N, M), lambda i: (i, 0, 0)),
            pl.BlockSpec((_BB, 1, N), lambda i: (i, 0, 0)),
            pl.BlockSpec((_BB, 1, N), lambda i: (i, 0, 0)),
            pl.BlockSpec((1, _BB, H), lambda i: (0, i, 0)),
            pl.BlockSpec((3 * H, M), lambda i: (0, 0)),
            pl.BlockSpec((3 * H, H), lambda i: (0, 0)),
            pl.BlockSpec((1, 3 * H), lambda i: (0, 0)),
            pl.BlockSpec((1, 3 * H), lambda i: (0, 0)),
            pl.BlockSpec((Pp, H), lambda i: (0, 0)),
            pl.BlockSpec((1, P), lambda i: (0, 0)),
        ],
        out_specs=[
            pl.BlockSpec((_BB, M), lambda i: (i, 0)),
            pl.BlockSpec((_BB, N, M), lambda i: (i, 0, 0)),
            pl.BlockSpec((_BB, 1, N), lambda i: (i, 0, 0)),
            pl.BlockSpec((_BB, 1, N), lambda i: (i, 0, 0)),
        ],
        out_shape=[
            jax.ShapeDtypeStruct((B, M), jnp.float32),
            jax.ShapeDtypeStruct((B, N, M), jnp.float32),
            jax.ShapeDtypeStruct((B, 1, N), jnp.float32),
            jax.ShapeDtypeStruct((B, 1, N), jnp.float32),
        ],
        scratch_shapes=[
            pltpu.VMEM((M, 3 * H), jnp.float32),
            pltpu.VMEM((H, 3 * H), jnp.float32),
            pltpu.VMEM((H, PP), jnp.float32),
            pltpu.VMEM((1, PP), jnp.float32),
        ],
        compiler_params=pltpu.CompilerParams(
            dimension_semantics=("arbitrary",),
            vmem_limit_bytes=56 * 1024 * 1024,
        ),
    )(x, memory, read_w, write_w, hidden, w_ih, w_hh,
      jnp.reshape(b_ih, (1, 3 * H)), jnp.reshape(b_hh, (1, 3 * H)),
      pwin, pbin)

    return y, new_mem, nrw, wtw


# speedup vs baseline: 1.1673x; 1.0113x over previous
import functools

import jax
import jax.numpy as jnp
from jax.experimental import pallas as pl
from jax.experimental.pallas import tpu as pltpu

_EPS = 1e-12
_BB = 8


def _address(cos, beta, gate, shift, gamma, prev_w):
    b = jax.nn.softplus(beta)
    a = b * cos
    a = a - jnp.max(a, axis=-1, keepdims=True)
    e = jnp.exp(a)
    wt = e / jnp.sum(e, axis=-1, keepdims=True)
    g = jax.nn.sigmoid(gate)
    wt = g * wt + (1.0 - g) * prev_w
    s = shift - jnp.max(shift, axis=-1, keepdims=True)
    s = jnp.exp(s)
    s = s / jnp.sum(s, axis=-1, keepdims=True)
    wm1 = jnp.concatenate([wt[:, 1:], wt[:, :1]], axis=1)
    wp1 = jnp.concatenate([wt[:, -1:], wt[:, :-1]], axis=1)
    wt = s[:, 0:1] * wm1 + s[:, 1:2] * wt + s[:, 2:3] * wp1
    gam = 1.0 + jax.nn.softplus(gamma)
    wt = jnp.exp(gam * jnp.log(wt + _EPS))
    return wt / (jnp.sum(wt, axis=-1, keepdims=True) + _EPS)


def _ntm_body(x_ref, mem_ref, rw_ref, ww_ref, h_ref, wih_ref, whh_ref,
              bih_ref, bhh_ref, pwin_ref, pbin_ref,
              y_ref, nm_ref, nrw_ref, wtw_ref,
              wiht_s, whht_s, pw_s, pb_s, *, H, PP):
    M = x_ref.shape[1]
    Bb = x_ref.shape[0]
    rl = M + 6
    P = 5 * M + 12

    @pl.when(pl.program_id(0) == 0)
    def _prep():
        wiht_s[...] = jnp.transpose(wih_ref[...])
        whht_s[...] = jnp.transpose(whh_ref[...])
        pwin = pwin_ref[...]
        t = jnp.concatenate([
            pwin[0:M], pwin[rl:rl + M],
            pwin[rl + M + 6:rl + 2 * M + 6],
            pwin[rl + 2 * M + 6:rl + 3 * M + 6],
            pwin[rl + 3 * M + 6:rl + 4 * M + 6],
            pwin[M:rl], pwin[rl + M:rl + M + 6],
            jnp.zeros((PP - P, H), jnp.float32)], axis=0)
        pw_s[...] = jnp.transpose(t)
        pbrow = pbin_ref[...]
        pb_s[...] = jnp.concatenate([
            pbrow[:, 0:M], pbrow[:, rl:rl + M],
            pbrow[:, rl + M + 6:rl + 2 * M + 6],
            pbrow[:, rl + 2 * M + 6:rl + 3 * M + 6],
            pbrow[:, rl + 3 * M + 6:rl + 4 * M + 6],
            pbrow[:, M:rl], pbrow[:, rl + M:rl + M + 6],
            jnp.zeros((1, PP - P), jnp.float32)], axis=1)

    mem = mem_ref[...]
    rw = rw_ref[:, 0, :]
    ww = ww_ref[:, 0, :]

    read_vec = jnp.concatenate(
        [jnp.dot(rw[b:b + 1, :], mem[b]) for b in range(Bb)], axis=0)

    h = h_ref[0]
    wih = wiht_s[...]
    whh = whht_s[...]
    bih = bih_ref[...][None]
    bhh = bhh_ref[...][None]

    def gru(xt, h):
        gi = jnp.dot(xt, wih) + bih
        gh = jnp.dot(h, whh) + bhh
        r = jax.nn.sigmoid(gi[:, :H] + gh[:, :H])
        z = jax.nn.sigmoid(gi[:, H:2 * H] + gh[:, H:2 * H])
        n = jnp.tanh(gi[:, 2 * H:] + r * gh[:, 2 * H:])
        return (1.0 - z) * n + z * h

    h = gru(x_ref[...], h)
    h = gru(read_vec, h)
    co = jnp.dot(h, pw_s[...]) + pb_s[...]

    key_r = co[:, :M]
    key_w = co[:, M:2 * M]
    erase = jax.nn.sigmoid(co[:, 2 * M:3 * M])
    addv = co[:, 3 * M:4 * M]
    y_ref[...] = co[:, 4 * M:5 * M]
    sc = co[:, 5 * M:5 * M + 12]

    inv_norm = 1.0 / (jnp.sqrt(jnp.sum(mem * mem, axis=2)) + _EPS)
    kr = key_r / (jnp.sqrt(jnp.sum(key_r * key_r, axis=1, keepdims=True)) + _EPS)
    kw = key_w / (jnp.sqrt(jnp.sum(key_w * key_w, axis=1, keepdims=True)) + _EPS)

    dcols = []
    for b in range(Bb):
        keys2 = jnp.concatenate([kr[b:b + 1, :], kw[b:b + 1, :]], axis=0)
        dcols.append(jax.lax.dot_general(
            mem[b], keys2, (((1,), (1,)), ((), ()))))
    d3 = jnp.swapaxes(jnp.stack(dcols, axis=0), 1, 2)
    cos_r = d3[:, 0, :] * inv_norm
    cos_w = d3[:, 1, :] * inv_norm

    new_rw = _address(cos_r, sc[:, 0:1], sc[:, 1:2], sc[:, 2:5], sc[:, 5:6], rw)
    wt_w = _address(cos_w, sc[:, 6:7], sc[:, 7:8], sc[:, 8:11], sc[:, 11:12], ww)
    nrw_ref[...] = new_rw[:, None, :]
    wtw_ref[...] = wt_w[:, None, :]

    nm_ref[...] = mem + wt_w[:, :, None] * (addv[:, None, :]
                                            - erase[:, None, :] * mem)


def kernel(x, memory, read_w, write_w, hidden, w_ih, w_hh, b_ih, b_hh,
           proj_w, proj_b):
    B, N, M = memory.shape
    H = hidden.shape[-1]
    P = 5 * M + 12
    PP = ((P + 127) // 128) * 128
    Pp = ((P + 7) // 8) * 8
    pwin = jnp.pad(proj_w, ((0, Pp - P), (0, 0)))
    pbin = jnp.reshape(proj_b, (1, P))

    grid = (B // _BB,)
    body = functools.partial(_ntm_body, H=H, PP=PP)
    y, new_mem, nrw, wtw = pl.pallas_call(
        body,
        grid=grid,
        in_specs=[
            pl.BlockSpec((_BB, M), lambda i: (i, 0)),
            pl.BlockSpec((_BB, N, M), lambda i: (i, 0, 0)),
            pl.BlockSpec((_BB, 1, N), lambda i: (i, 0, 0)),
            pl.BlockSpec((_BB, 1, N), lambda i: (i, 0, 0)),
            pl.BlockSpec((1, _BB, H), lambda i: (0, i, 0)),
            pl.BlockSpec((3 * H, M), lambda i: (0, 0)),
            pl.BlockSpec((3 * H, H), lambda i: (0, 0)),
            pl.BlockSpec((3 * H,), lambda i: (0,)),
            pl.BlockSpec((3 * H,), lambda i: (0,)),
            pl.BlockSpec((Pp, H), lambda i: (0, 0)),
            pl.BlockSpec((1, P), lambda i: (0, 0)),
        ],
        out_specs=[
            pl.BlockSpec((_BB, M), lambda i: (i, 0)),
            pl.BlockSpec((_BB, N, M), lambda i: (i, 0, 0)),
            pl.BlockSpec((_BB, 1, N), lambda i: (i, 0, 0)),
            pl.BlockSpec((_BB, 1, N), lambda i: (i, 0, 0)),
        ],
        out_shape=[
            jax.ShapeDtypeStruct((B, M), jnp.float32),
            jax.ShapeDtypeStruct((B, N, M), jnp.float32),
            jax.ShapeDtypeStruct((B, 1, N), jnp.float32),
            jax.ShapeDtypeStruct((B, 1, N), jnp.float32),
        ],
        scratch_shapes=[
            pltpu.VMEM((M, 3 * H), jnp.float32),
            pltpu.VMEM((H, 3 * H), jnp.float32),
            pltpu.VMEM((H, PP), jnp.float32),
            pltpu.VMEM((1, PP), jnp.float32),
        ],
        compiler_params=pltpu.CompilerParams(
            dimension_semantics=("arbitrary",),
            vmem_limit_bytes=56 * 1024 * 1024,
        ),
    )(x, memory, read_w, write_w, hidden, w_ih, w_hh,
      b_ih, b_hh,
      pwin, pbin)

    return y, new_mem, nrw, wtw


# speedup vs baseline: 1.1942x; 1.0231x over previous
import functools

import jax
import jax.numpy as jnp
from jax.experimental import pallas as pl
from jax.experimental.pallas import tpu as pltpu

_EPS = 1e-12
_BB = 8


def _address(cos, beta, gate, shift, gamma, prev_w):
    b = jax.nn.softplus(beta)
    a = b * cos
    a = a - jnp.max(a, axis=-1, keepdims=True)
    e = jnp.exp(a)
    wt = e / jnp.sum(e, axis=-1, keepdims=True)
    g = jax.nn.sigmoid(gate)
    wt = g * wt + (1.0 - g) * prev_w
    s = shift - jnp.max(shift, axis=-1, keepdims=True)
    s = jnp.exp(s)
    s = s / jnp.sum(s, axis=-1, keepdims=True)
    wm1 = jnp.concatenate([wt[:, 1:], wt[:, :1]], axis=1)
    wp1 = jnp.concatenate([wt[:, -1:], wt[:, :-1]], axis=1)
    wt = s[:, 0:1] * wm1 + s[:, 1:2] * wt + s[:, 2:3] * wp1
    gam = 1.0 + jax.nn.softplus(gamma)
    wt = jnp.exp(gam * jnp.log(wt + _EPS))
    return wt / (jnp.sum(wt, axis=-1, keepdims=True) + _EPS)


def _ntm_body(x_ref, mem_ref, rw_ref, ww_ref, h_ref, wih_ref, whh_ref,
              bih_ref, bhh_ref, pwin_ref, pbin_ref,
              y_ref, nm_ref, nrw_ref, wtw_ref,
              wiht_s, whht_s, pw_s, pb_s, *, H, PP):
    M = x_ref.shape[1]
    Bb = x_ref.shape[0]
    rl = M + 6
    P = 5 * M + 12

    @pl.when(pl.program_id(0) == 0)
    def _prep():
        wiht_s[...] = jnp.transpose(wih_ref[...])
        whht_s[...] = jnp.transpose(whh_ref[...])
        pwin = pwin_ref[...]
        t = jnp.concatenate([
            pwin[0:M], pwin[rl:rl + M],
            pwin[rl + M + 6:rl + 2 * M + 6],
            pwin[rl + 2 * M + 6:rl + 3 * M + 6],
            pwin[rl + 3 * M + 6:rl + 4 * M + 6],
            pwin[M:rl], pwin[rl + M:rl + M + 6],
            jnp.zeros((PP - P, H), jnp.float32)], axis=0)
        pw_s[...] = jnp.transpose(t)
        pbrow = pbin_ref[...][None]
        pb_s[...] = jnp.concatenate([
            pbrow[:, 0:M], pbrow[:, rl:rl + M],
            pbrow[:, rl + M + 6:rl + 2 * M + 6],
            pbrow[:, rl + 2 * M + 6:rl + 3 * M + 6],
            pbrow[:, rl + 3 * M + 6:rl + 4 * M + 6],
            pbrow[:, M:rl], pbrow[:, rl + M:rl + M + 6],
            jnp.zeros((1, PP - P), jnp.float32)], axis=1)

    mem = mem_ref[...]
    rw = rw_ref[:, 0, :]
    ww = ww_ref[:, 0, :]

    read_vec = jnp.concatenate(
        [jnp.dot(rw[b:b + 1, :], mem[b]) for b in range(Bb)], axis=0)

    h = h_ref[0]
    wih = wiht_s[...]
    whh = whht_s[...]
    bih = bih_ref[...][None]
    bhh = bhh_ref[...][None]

    def gru(xt, h):
        gi = jnp.dot(xt, wih) + bih
        gh = jnp.dot(h, whh) + bhh
        r = jax.nn.sigmoid(gi[:, :H] + gh[:, :H])
        z = jax.nn.sigmoid(gi[:, H:2 * H] + gh[:, H:2 * H])
        n = jnp.tanh(gi[:, 2 * H:] + r * gh[:, 2 * H:])
        return (1.0 - z) * n + z * h

    h = gru(x_ref[...], h)
    h = gru(read_vec, h)
    co = jnp.dot(h, pw_s[...]) + pb_s[...]

    key_r = co[:, :M]
    key_w = co[:, M:2 * M]
    erase = jax.nn.sigmoid(co[:, 2 * M:3 * M])
    addv = co[:, 3 * M:4 * M]
    y_ref[...] = co[:, 4 * M:5 * M]
    sc = co[:, 5 * M:5 * M + 12]

    inv_norm = 1.0 / (jnp.sqrt(jnp.sum(mem * mem, axis=2)) + _EPS)
    kr = key_r / (jnp.sqrt(jnp.sum(key_r * key_r, axis=1, keepdims=True)) + _EPS)
    kw = key_w / (jnp.sqrt(jnp.sum(key_w * key_w, axis=1, keepdims=True)) + _EPS)

    dcols = []
    for b in range(Bb):
        keys2 = jnp.concatenate([kr[b:b + 1, :], kw[b:b + 1, :]], axis=0)
        dcols.append(jax.lax.dot_general(
            mem[b], keys2, (((1,), (1,)), ((), ()))))
    d3 = jnp.swapaxes(jnp.stack(dcols, axis=0), 1, 2)
    cos_r = d3[:, 0, :] * inv_norm
    cos_w = d3[:, 1, :] * inv_norm

    new_rw = _address(cos_r, sc[:, 0:1], sc[:, 1:2], sc[:, 2:5], sc[:, 5:6], rw)
    wt_w = _address(cos_w, sc[:, 6:7], sc[:, 7:8], sc[:, 8:11], sc[:, 11:12], ww)
    nrw_ref[...] = new_rw[:, None, :]
    wtw_ref[...] = wt_w[:, None, :]

    nm_ref[...] = mem + wt_w[:, :, None] * (addv[:, None, :]
                                            - erase[:, None, :] * mem)


def kernel(x, memory, read_w, write_w, hidden, w_ih, w_hh, b_ih, b_hh,
           proj_w, proj_b):
    B, N, M = memory.shape
    H = hidden.shape[-1]
    P = 5 * M + 12
    PP = ((P + 127) // 128) * 128
    Pp = ((P + 7) // 8) * 8
    pwin, pbin = proj_w, proj_b

    grid = (B // _BB,)
    body = functools.partial(_ntm_body, H=H, PP=PP)
    y, new_mem, nrw, wtw = pl.pallas_call(
        body,
        grid=grid,
        in_specs=[
            pl.BlockSpec((_BB, M), lambda i: (i, 0)),
            pl.BlockSpec((_BB, N, M), lambda i: (i, 0, 0)),
            pl.BlockSpec((_BB, 1, N), lambda i: (i, 0, 0)),
            pl.BlockSpec((_BB, 1, N), lambda i: (i, 0, 0)),
            pl.BlockSpec((1, _BB, H), lambda i: (0, i, 0)),
            pl.BlockSpec((3 * H, M), lambda i: (0, 0)),
            pl.BlockSpec((3 * H, H), lambda i: (0, 0)),
            pl.BlockSpec((3 * H,), lambda i: (0,)),
            pl.BlockSpec((3 * H,), lambda i: (0,)),
            pl.BlockSpec((P, H), lambda i: (0, 0)),
            pl.BlockSpec((P,), lambda i: (0,)),
        ],
        out_specs=[
            pl.BlockSpec((_BB, M), lambda i: (i, 0)),
            pl.BlockSpec((_BB, N, M), lambda i: (i, 0, 0)),
            pl.BlockSpec((_BB, 1, N), lambda i: (i, 0, 0)),
            pl.BlockSpec((_BB, 1, N), lambda i: (i, 0, 0)),
        ],
        out_shape=[
            jax.ShapeDtypeStruct((B, M), jnp.float32),
            jax.ShapeDtypeStruct((B, N, M), jnp.float32),
            jax.ShapeDtypeStruct((B, 1, N), jnp.float32),
            jax.ShapeDtypeStruct((B, 1, N), jnp.float32),
        ],
        scratch_shapes=[
            pltpu.VMEM((M, 3 * H), jnp.float32),
            pltpu.VMEM((H, 3 * H), jnp.float32),
            pltpu.VMEM((H, PP), jnp.float32),
            pltpu.VMEM((1, PP), jnp.float32),
        ],
        compiler_params=pltpu.CompilerParams(
            dimension_semantics=("arbitrary",),
            vmem_limit_bytes=56 * 1024 * 1024,
        ),
    )(x, memory, read_w, write_w, hidden, w_ih, w_hh,
      b_ih, b_hh,
      pwin, pbin)

    return y, new_mem, nrw, wtw
